# Initial kernel scaffold; baseline (speedup 1.0000x reference)
import jax
import jax.numpy as jnp
from jax.experimental import pallas as pl


def kernel(scores):
    raise NotImplementedError("write your pallas kernel here")


# speedup vs baseline: 1.0004x; 1.0004x over previous
import jax
import jax.numpy as jnp
from jax.experimental import pallas as pl

_NUCLEUS_P = 0.9
_N = 1_000_000
_R = 1024
_C = 1024
_PAD = _R * _C - _N


def _nucleus_body(sp_ref, od_ref, g_ref, out_ref, sel_ref):
    sp = sp_ref[0]
    g = g_ref[0]
    ii = jax.lax.broadcasted_iota(jnp.int32, (_R, _C), 0)
    jj = jax.lax.broadcasted_iota(jnp.int32, (_R, _C), 1)
    upper = (ii <= jj).astype(jnp.float32)
    within = jax.lax.dot_general(
        sp, upper, (((1,), (0,)), ((), ())),
        precision=jax.lax.Precision.HIGHEST)
    totals = within[:, _C - 1:_C]
    lower = (ii > jj).astype(jnp.float32)
    carry = jax.lax.dot_general(
        lower, totals, (((1,), (0,)), ((), ())),
        precision=jax.lax.Precision.HIGHEST)
    csum = within + carry

    flat = ii * _C + jj
    mask = (csum <= _NUCLEUS_P) | (flat == 0)
    kept = jnp.where(mask, sp, 0.0)
    z = jnp.sum(kept)
    out = kept / z
    out_ref[0] = out

    logp = jnp.where(mask, jnp.log(jnp.maximum(out, 1e-30)), -jnp.inf)
    y = logp + g
    m = jnp.max(y)
    first = jnp.min(jnp.where(y == m, flat, jnp.int32(2**30)))
    sel = jnp.max(jnp.where(flat == first, od_ref[0], jnp.int32(-(2**30))))
    sel_ref[0] = jnp.full((1, 128), sel, jnp.int32)


def kernel(scores):
    probs = jax.nn.softmax(scores, axis=-1)
    order = jnp.argsort(-probs, axis=-1)
    sorted_probs = jnp.take_along_axis(probs, order, axis=-1)
    gumbel = jax.random.gumbel(jax.random.key(42), scores.shape, scores.dtype)

    b = scores.shape[0]
    sp = jnp.pad(sorted_probs, ((0, 0), (0, _PAD))).reshape(b, _R, _C)
    od = jnp.pad(order, ((0, 0), (0, _PAD))).reshape(b, _R, _C)
    gp = jnp.pad(gumbel, ((0, 0), (0, _PAD)),
                 constant_values=-jnp.inf).reshape(b, _R, _C)

    out, sel = pl.pallas_call(
        _nucleus_body,
        grid=(b,),
        in_specs=[
            pl.BlockSpec((1, _R, _C), lambda i: (i, 0, 0)),
            pl.BlockSpec((1, _R, _C), lambda i: (i, 0, 0)),
            pl.BlockSpec((1, _R, _C), lambda i: (i, 0, 0)),
        ],
        out_specs=[
            pl.BlockSpec((1, _R, _C), lambda i: (i, 0, 0)),
            pl.BlockSpec((1, 1, 128), lambda i: (i, 0, 0)),
        ],
        out_shape=[
            jax.ShapeDtypeStruct((b, _R, _C), jnp.float32),
            jax.ShapeDtypeStruct((b, 1, 128), jnp.int32),
        ],
    )(sp, od, gp)

    nucleus_probs = out.reshape(b, _R * _C)[:, :_N]
    return sel[:, 0, 0], nucleus_probs


# speedup vs baseline: 1.1515x; 1.1511x over previous
import jax
import jax.numpy as jnp
from jax.experimental import pallas as pl

_NUCLEUS_P = 0.9
_N = 1_000_000
_R = 1024
_C = 1024
_PAD = _R * _C - _N


def _nucleus_body(sp_ref, od_ref, g_ref, out_ref, sel_ref):
    sp = sp_ref[0]
    g = g_ref[0]
    ii = jax.lax.broadcasted_iota(jnp.int32, (_R, _C), 0)
    jj = jax.lax.broadcasted_iota(jnp.int32, (_R, _C), 1)
    upper = (ii <= jj).astype(jnp.float32)
    within = jax.lax.dot_general(
        sp, upper, (((1,), (0,)), ((), ())),
        precision=jax.lax.Precision.HIGHEST)
    totals = within[:, _C - 1:_C]
    lower = (ii > jj).astype(jnp.float32)
    carry = jax.lax.dot_general(
        lower, totals, (((1,), (0,)), ((), ())),
        precision=jax.lax.Precision.HIGHEST)
    csum = within + carry

    flat = ii * _C + jj
    mask = (csum <= _NUCLEUS_P) | (flat == 0)
    kept = jnp.where(mask, sp, 0.0)
    z = jnp.sum(kept)
    out = kept / z
    out_ref[0] = out

    logp = jnp.where(mask, jnp.log(jnp.maximum(out, 1e-30)), -jnp.inf)
    y = logp + g
    m = jnp.max(y)
    first = jnp.min(jnp.where(y == m, flat, jnp.int32(2**30)))
    sel = jnp.max(jnp.where(flat == first, od_ref[0], jnp.int32(-(2**30))))
    sel_ref[0] = jnp.full((1, 128), sel, jnp.int32)


def kernel(scores):
    probs = jax.nn.softmax(scores, axis=-1)
    iota = jax.lax.broadcasted_iota(jnp.int32, probs.shape, 1)
    neg_sorted, order = jax.lax.sort((-probs, iota), num_keys=1, is_stable=True)
    sorted_probs = -neg_sorted
    gumbel = jax.random.gumbel(jax.random.key(42), scores.shape, scores.dtype)

    b = scores.shape[0]
    sp = jnp.pad(sorted_probs, ((0, 0), (0, _PAD))).reshape(b, _R, _C)
    od = jnp.pad(order, ((0, 0), (0, _PAD))).reshape(b, _R, _C)
    gp = jnp.pad(gumbel, ((0, 0), (0, _PAD)),
                 constant_values=-jnp.inf).reshape(b, _R, _C)

    out, sel = pl.pallas_call(
        _nucleus_body,
        grid=(b,),
        in_specs=[
            pl.BlockSpec((1, _R, _C), lambda i: (i, 0, 0)),
            pl.BlockSpec((1, _R, _C), lambda i: (i, 0, 0)),
            pl.BlockSpec((1, _R, _C), lambda i: (i, 0, 0)),
        ],
        out_specs=[
            pl.BlockSpec((1, _R, _C), lambda i: (i, 0, 0)),
            pl.BlockSpec((1, 1, 128), lambda i: (i, 0, 0)),
        ],
        out_shape=[
            jax.ShapeDtypeStruct((b, _R, _C), jnp.float32),
            jax.ShapeDtypeStruct((b, 1, 128), jnp.int32),
        ],
    )(sp, od, gp)

    nucleus_probs = out.reshape(b, _R * _C)[:, :_N]
    return sel[:, 0, 0], nucleus_probs
